# baseline (speedup 1.0000x reference)
import functools

import jax
import jax.numpy as jnp
from jax.experimental import pallas as pl
from jax.experimental.pallas import tpu as pltpu

_B = 16
_N = 5000
_M = 8192


def _sort_body(prob_ref, cx_ref, cy_ref, w_ref, h_ref, sw_ref, sh_ref,
               scores_ref, x1_ref, y1_ref, x2_ref, y2_ref):
    prob = prob_ref[...]
    key = jax.lax.bitcast_convert_type(prob, jnp.int32)
    iota = jax.lax.broadcasted_iota(jnp.int32, (_B, _M), 1)
    idx = iota

    iw = sw_ref[...]
    ih = sh_ref[...]
    cx = cx_ref[...]
    cy = cy_ref[...]
    w = w_ref[...]
    h = h_ref[...]
    x1 = (cx - 0.5 * w) * iw
    y1 = (cy - 0.5 * h) * ih
    x2 = (cx + 0.5 * w) * iw
    y2 = (cy + 0.5 * h) * ih

    def one_pass(asc, d, arrs):
        is_hi = (iota & d) != 0
        flip = jnp.logical_xor(is_hi, asc)
        partners = [
            jnp.where(is_hi, pltpu.roll(a, d, 1), pltpu.roll(a, _M - d, 1))
            for a in arrs
        ]
        k, i = arrs[0], arrs[1]
        kp, ip = partners[0], partners[1]
        before = (k > kp) | ((k == kp) & (i < ip))
        take_mine = jnp.logical_xor(before, flip)
        return [jnp.where(take_mine, a, p) for a, p in zip(arrs, partners)]

    def stage(s, arrs):
        size = jax.lax.shift_left(2, s)
        asc = (iota & size) != 0

        def substage(j, arrs):
            d = jax.lax.shift_right_logical(size, j + 1)
            return one_pass(asc, d, arrs)

        return jax.lax.fori_loop(0, s + 1, substage, arrs)

    arrs = jax.lax.fori_loop(0, 13, stage, [key, idx, x1, y1, x2, y2])

    scores_ref[...] = jax.lax.bitcast_convert_type(arrs[0], jnp.float32)[:, :_N]
    x1_ref[...] = arrs[2][:, :_N]
    y1_ref[...] = arrs[3][:, :_N]
    x2_ref[...] = arrs[4][:, :_N]
    y2_ref[...] = arrs[5][:, :_N]


@functools.partial(jax.jit, static_argnames=())
def kernel(pred_logits, pred_boxes, target_sizes):
    B, N, C = pred_logits.shape
    assert (B, N, C) == (_B, _N, 1)

    prob = jax.nn.sigmoid(pred_logits.reshape(B, N))
    pad = _M - N
    prob_p = jnp.pad(prob, ((0, 0), (0, pad)), constant_values=-1.0)

    cx = jnp.pad(pred_boxes[:, :, 0], ((0, 0), (0, pad)))
    cy = jnp.pad(pred_boxes[:, :, 1], ((0, 0), (0, pad)))
    w = jnp.pad(pred_boxes[:, :, 2], ((0, 0), (0, pad)))
    h = jnp.pad(pred_boxes[:, :, 3], ((0, 0), (0, pad)))

    img_h = target_sizes[:, 0].astype(jnp.float32)[:, None]
    img_w = target_sizes[:, 1].astype(jnp.float32)[:, None]

    out_shape = [
        jax.ShapeDtypeStruct((B, N), jnp.float32),
        jax.ShapeDtypeStruct((B, N), jnp.float32),
        jax.ShapeDtypeStruct((B, N), jnp.float32),
        jax.ShapeDtypeStruct((B, N), jnp.float32),
        jax.ShapeDtypeStruct((B, N), jnp.float32),
    ]
    scores, x1, y1, x2, y2 = pl.pallas_call(
        _sort_body,
        out_shape=out_shape,
    )(prob_p, cx, cy, w, h, img_w, img_h)

    boxes = jnp.stack([x1, y1, x2, y2], axis=-1)
    labels = jnp.full((B, N), 1, dtype=jnp.int32)
    return scores, labels, boxes


# speedup vs baseline: 1.7621x; 1.7621x over previous
import functools

import jax
import jax.numpy as jnp
from jax.experimental import pallas as pl
from jax.experimental.pallas import tpu as pltpu

_B = 16
_N = 5000
_M = 8192
_R = 64
_C = 128


def _sort_body(prob_ref, cx_ref, cy_ref, w_ref, h_ref, sw_ref, sh_ref,
               scores_ref, x1_ref, y1_ref, x2_ref, y2_ref):
    shp = (1, _R, _C)
    r_iota = jax.lax.broadcasted_iota(jnp.int32, shp, 1)
    c_iota = jax.lax.broadcasted_iota(jnp.int32, shp, 2)
    lin = r_iota * _C + c_iota

    prob = prob_ref[...]
    key = jax.lax.bitcast_convert_type(prob, jnp.int32)
    idx = lin

    iw = sw_ref[...]
    ih = sh_ref[...]
    cx = cx_ref[...]
    cy = cy_ref[...]
    w = w_ref[...]
    h = h_ref[...]
    x1 = (cx - 0.5 * w) * iw
    y1 = (cy - 0.5 * h) * ih
    x2 = (cx + 0.5 * w) * iw
    y2 = (cy + 0.5 * h) * ih

    def lane_pass(arrs, asc, d):
        is_hi = (c_iota & d) != 0
        flip = jnp.logical_xor(is_hi, asc)
        partners = [
            jnp.where(is_hi, pltpu.roll(a, d, 2), pltpu.roll(a, _C - d, 2))
            for a in arrs
        ]
        k, i = arrs[0], arrs[1]
        kp, ip = partners[0], partners[1]
        before = (k > kp) | ((k == kp) & (i < ip))
        take_mine = jnp.logical_xor(before, flip)
        return [jnp.where(take_mine, a, p) for a, p in zip(arrs, partners)]

    def row_pass(arrs, asc, dr):
        is_hi = (r_iota & dr) != 0
        flip = jnp.logical_xor(is_hi, asc)
        partners = [
            jnp.where(is_hi, pltpu.roll(a, dr, 1), pltpu.roll(a, _R - dr, 1))
            for a in arrs
        ]
        k, i = arrs[0], arrs[1]
        kp, ip = partners[0], partners[1]
        before = (k > kp) | ((k == kp) & (i < ip))
        take_mine = jnp.logical_xor(before, flip)
        return [jnp.where(take_mine, a, p) for a, p in zip(arrs, partners)]

    arrs = [key, idx, x1, y1, x2, y2]

    def stage_a(s, arrs):
        size = jax.lax.shift_left(2, s)
        asc = (lin & size) != 0

        def sub(j, arrs):
            d = jax.lax.shift_right_logical(size, j + 1)
            return lane_pass(arrs, asc, d)

        return jax.lax.fori_loop(0, s + 1, sub, arrs)

    arrs = jax.lax.fori_loop(0, 7, stage_a, arrs)

    def stage_b(s, arrs):
        size = jax.lax.shift_left(2, s)
        asc = (lin & size) != 0
        dr0 = jax.lax.shift_right_logical(size, 8)

        def sub_rows(j, arrs):
            dr = jax.lax.shift_right_logical(dr0, j)
            return row_pass(arrs, asc, dr)

        arrs = jax.lax.fori_loop(0, s - 6, sub_rows, arrs)

        def sub_lanes(j, arrs):
            d = jax.lax.shift_right_logical(64, j)
            return lane_pass(arrs, asc, d)

        return jax.lax.fori_loop(0, 7, sub_lanes, arrs)

    arrs = jax.lax.fori_loop(7, 13, stage_b, arrs)

    scores_ref[...] = jax.lax.bitcast_convert_type(arrs[0], jnp.float32)
    x1_ref[...] = arrs[2]
    y1_ref[...] = arrs[3]
    x2_ref[...] = arrs[4]
    y2_ref[...] = arrs[5]


@functools.partial(jax.jit, static_argnames=())
def kernel(pred_logits, pred_boxes, target_sizes):
    B, N, C = pred_logits.shape
    assert (B, N, C) == (_B, _N, 1)

    prob = jax.nn.sigmoid(pred_logits.reshape(B, N))
    pad = _M - N
    prob_p = jnp.pad(prob, ((0, 0), (0, pad)), constant_values=-1.0).reshape(B, _R, _C)

    cx = jnp.pad(pred_boxes[:, :, 0], ((0, 0), (0, pad))).reshape(B, _R, _C)
    cy = jnp.pad(pred_boxes[:, :, 1], ((0, 0), (0, pad))).reshape(B, _R, _C)
    w = jnp.pad(pred_boxes[:, :, 2], ((0, 0), (0, pad))).reshape(B, _R, _C)
    h = jnp.pad(pred_boxes[:, :, 3], ((0, 0), (0, pad))).reshape(B, _R, _C)

    img_h = target_sizes[:, 0].astype(jnp.float32)
    img_w = target_sizes[:, 1].astype(jnp.float32)
    sw = jnp.broadcast_to(img_w[:, None, None], (B, 1, _C))
    sh = jnp.broadcast_to(img_h[:, None, None], (B, 1, _C))

    blk = pl.BlockSpec((1, _R, _C), lambda b: (b, 0, 0))
    sblk = pl.BlockSpec((1, 1, _C), lambda b: (b, 0, 0))
    out_shape = [jax.ShapeDtypeStruct((B, _R, _C), jnp.float32)] * 5

    scores, x1, y1, x2, y2 = pl.pallas_call(
        _sort_body,
        grid=(B,),
        in_specs=[blk, blk, blk, blk, blk, sblk, sblk],
        out_specs=[blk] * 5,
        out_shape=out_shape,
    )(prob_p, cx, cy, w, h, sw, sh)

    scores = scores.reshape(B, _M)[:, :_N]
    boxes = jnp.stack(
        [a.reshape(B, _M)[:, :_N] for a in (x1, y1, x2, y2)], axis=-1)
    labels = jnp.full((B, N), 1, dtype=jnp.int32)
    return scores, labels, boxes


# speedup vs baseline: 2.3830x; 1.3524x over previous
import functools

import jax
import jax.numpy as jnp
from jax import lax
from jax.experimental import pallas as pl
from jax.experimental.pallas import tpu as pltpu
from jax.experimental.pallas import tpu_sc as plsc

_B = 16
_N = 5000
_M = 8192
_R = 64
_C = 128

_NW = 32
_CH = (_B * _M) // _NW


def _sort_body(prob_ref, cx_ref, cy_ref, w_ref, h_ref, sw_ref, sh_ref,
               scores_ref, gidx_ref, x1_ref, y1_ref, x2_ref, y2_ref):
    shp = (1, _R, _C)
    r_iota = jax.lax.broadcasted_iota(jnp.int32, shp, 1)
    c_iota = jax.lax.broadcasted_iota(jnp.int32, shp, 2)
    lin = r_iota * _C + c_iota

    prob = prob_ref[...]
    key = jax.lax.bitcast_convert_type(prob, jnp.int32)
    idx = lin

    iw = sw_ref[...]
    ih = sh_ref[...]
    cx = cx_ref[...]
    cy = cy_ref[...]
    w = w_ref[...]
    h = h_ref[...]
    x1_ref[...] = (cx - 0.5 * w) * iw
    y1_ref[...] = (cy - 0.5 * h) * ih
    x2_ref[...] = (cx + 0.5 * w) * iw
    y2_ref[...] = (cy + 0.5 * h) * ih

    def lane_pass(arrs, asc, d):
        is_hi = (c_iota & d) != 0
        flip = jnp.logical_xor(is_hi, asc)
        partners = [
            jnp.where(is_hi, pltpu.roll(a, d, 2), pltpu.roll(a, _C - d, 2))
            for a in arrs
        ]
        k, i = arrs
        kp, ip = partners
        before = (k > kp) | ((k == kp) & (i < ip))
        take_mine = jnp.logical_xor(before, flip)
        return [jnp.where(take_mine, a, p) for a, p in zip(arrs, partners)]

    def row_pass(arrs, asc, dr):
        is_hi = (r_iota & dr) != 0
        flip = jnp.logical_xor(is_hi, asc)
        partners = [
            jnp.where(is_hi, pltpu.roll(a, dr, 1), pltpu.roll(a, _R - dr, 1))
            for a in arrs
        ]
        k, i = arrs
        kp, ip = partners
        before = (k > kp) | ((k == kp) & (i < ip))
        take_mine = jnp.logical_xor(before, flip)
        return [jnp.where(take_mine, a, p) for a, p in zip(arrs, partners)]

    arrs = [key, idx]

    def stage_a(s, arrs):
        size = jax.lax.shift_left(2, s)
        asc = (lin & size) != 0

        def sub(j, arrs):
            d = jax.lax.shift_right_logical(size, j + 1)
            return lane_pass(arrs, asc, d)

        return jax.lax.fori_loop(0, s + 1, sub, arrs)

    arrs = jax.lax.fori_loop(0, 7, stage_a, arrs)

    def stage_b(s, arrs):
        size = jax.lax.shift_left(2, s)
        asc = (lin & size) != 0
        dr0 = jax.lax.shift_right_logical(size, 8)

        def sub_rows(j, arrs):
            dr = jax.lax.shift_right_logical(dr0, j)
            return row_pass(arrs, asc, dr)

        arrs = jax.lax.fori_loop(0, s - 6, sub_rows, arrs)

        def sub_lanes(j, arrs):
            d = jax.lax.shift_right_logical(64, j)
            return lane_pass(arrs, asc, d)

        return jax.lax.fori_loop(0, 7, sub_lanes, arrs)

    arrs = jax.lax.fori_loop(7, 13, stage_b, arrs)

    scores_ref[...] = jax.lax.bitcast_convert_type(arrs[0], jnp.float32)
    gidx_ref[...] = arrs[1] + pl.program_id(0) * _M


def _gather_body(idx_hbm, t0, t1, t2, t3, o0, o1, o2, o3,
                 idx_v, b0, b1, b2, b3, sem):
    wid = lax.axis_index("s") * 2 + lax.axis_index("c")
    base = wid * _CH
    pltpu.sync_copy(idx_hbm.at[pl.ds(base, _CH)], idx_v)
    cps = [
        pltpu.async_copy(t.at[idx_v], b, sem)
        for t, b in ((t0, b0), (t1, b1), (t2, b2), (t3, b3))
    ]
    for c in cps:
        c.wait()
    for b, o in ((b0, o0), (b1, o1), (b2, o2), (b3, o3)):
        pltpu.sync_copy(b, o.at[pl.ds(base, _CH)])


@functools.partial(jax.jit, static_argnames=())
def kernel(pred_logits, pred_boxes, target_sizes):
    B, N, C = pred_logits.shape
    assert (B, N, C) == (_B, _N, 1)

    prob = jax.nn.sigmoid(pred_logits.reshape(B, N))
    pad = _M - N
    prob_p = jnp.pad(prob, ((0, 0), (0, pad)), constant_values=-1.0).reshape(B, _R, _C)

    cx = jnp.pad(pred_boxes[:, :, 0], ((0, 0), (0, pad))).reshape(B, _R, _C)
    cy = jnp.pad(pred_boxes[:, :, 1], ((0, 0), (0, pad))).reshape(B, _R, _C)
    w = jnp.pad(pred_boxes[:, :, 2], ((0, 0), (0, pad))).reshape(B, _R, _C)
    h = jnp.pad(pred_boxes[:, :, 3], ((0, 0), (0, pad))).reshape(B, _R, _C)

    img_h = target_sizes[:, 0].astype(jnp.float32)
    img_w = target_sizes[:, 1].astype(jnp.float32)
    sw = jnp.broadcast_to(img_w[:, None, None], (B, 1, _C))
    sh = jnp.broadcast_to(img_h[:, None, None], (B, 1, _C))

    blk = pl.BlockSpec((1, _R, _C), lambda b: (b, 0, 0))
    sblk = pl.BlockSpec((1, 1, _C), lambda b: (b, 0, 0))
    out_shape = [
        jax.ShapeDtypeStruct((B, _R, _C), jnp.float32),
        jax.ShapeDtypeStruct((B, _R, _C), jnp.int32),
        jax.ShapeDtypeStruct((B, _R, _C), jnp.float32),
        jax.ShapeDtypeStruct((B, _R, _C), jnp.float32),
        jax.ShapeDtypeStruct((B, _R, _C), jnp.float32),
        jax.ShapeDtypeStruct((B, _R, _C), jnp.float32),
    ]
    scores, gidx, x1, y1, x2, y2 = pl.pallas_call(
        _sort_body,
        grid=(B,),
        in_specs=[blk, blk, blk, blk, blk, sblk, sblk],
        out_specs=[blk] * 6,
        out_shape=out_shape,
    )(prob_p, cx, cy, w, h, sw, sh)

    flat = (_B * _M,)
    mesh = plsc.VectorSubcoreMesh(core_axis_name="c", subcore_axis_name="s")
    gathered = pl.kernel(
        _gather_body,
        mesh=mesh,
        out_type=[jax.ShapeDtypeStruct(flat, jnp.float32)] * 4,
        scratch_types=[
            pltpu.VMEM((_CH,), jnp.int32),
            pltpu.VMEM((_CH,), jnp.float32),
            pltpu.VMEM((_CH,), jnp.float32),
            pltpu.VMEM((_CH,), jnp.float32),
            pltpu.VMEM((_CH,), jnp.float32),
            pltpu.SemaphoreType.DMA,
        ],
    )(gidx.reshape(flat), x1.reshape(flat), y1.reshape(flat),
      x2.reshape(flat), y2.reshape(flat))

    scores = scores.reshape(B, _M)[:, :_N]
    boxes = jnp.stack(
        [g.reshape(B, _M)[:, :_N] for g in gathered], axis=-1)
    labels = jnp.full((B, N), 1, dtype=jnp.int32)
    return scores, labels, boxes


# speedup vs baseline: 3.3791x; 1.4180x over previous
import functools

import jax
import jax.numpy as jnp
from jax import lax
from jax.experimental import pallas as pl
from jax.experimental.pallas import tpu as pltpu
from jax.experimental.pallas import tpu_sc as plsc

_B = 16
_N = 5000
_M = 8192
_S = 2
_R = 64 * _S
_C = 128

_NW = 32
_CH = (_B * _M) // _NW


def _sort_body(prob_ref, cx_ref, cy_ref, w_ref, h_ref, sw_ref, sh_ref,
               scores_ref, gidx_ref, x1_ref, y1_ref, x2_ref, y2_ref):
    shp = (1, _R, _C)
    r_iota = jax.lax.broadcasted_iota(jnp.int32, shp, 1)
    c_iota = jax.lax.broadcasted_iota(jnp.int32, shp, 2)
    rloc = r_iota & 63
    lin = rloc * _C + c_iota

    prob = prob_ref[...]
    key = jax.lax.bitcast_convert_type(prob, jnp.int32)
    idx = lin

    iw = sw_ref[...]
    ih = sh_ref[...]
    cx = cx_ref[...]
    cy = cy_ref[...]
    w = w_ref[...]
    h = h_ref[...]
    x1_ref[...] = (cx - 0.5 * w) * iw
    y1_ref[...] = (cy - 0.5 * h) * ih
    x2_ref[...] = (cx + 0.5 * w) * iw
    y2_ref[...] = (cy + 0.5 * h) * ih

    def cmpx(arrs, asc, is_hi, partners):
        flip = jnp.logical_xor(is_hi, asc)
        k, i = arrs
        kp, ip = partners
        before = (k > kp) | ((k == kp) & (i < ip))
        take_mine = jnp.logical_xor(before, flip)
        return [jnp.where(take_mine, a, p) for a, p in zip(arrs, partners)]

    def lane_pass(arrs, asc, d):
        is_hi = (c_iota & d) != 0
        partners = [
            jnp.where(is_hi, pltpu.roll(a, d, 2), pltpu.roll(a, _C - d, 2))
            for a in arrs
        ]
        return cmpx(arrs, asc, is_hi, partners)

    def row_pass(arrs, asc, dr):
        is_hi = (r_iota & dr) != 0
        partners = [
            jnp.where(is_hi, pltpu.roll(a, dr, 1), pltpu.roll(a, _R - dr, 1))
            for a in arrs
        ]
        return cmpx(arrs, asc, is_hi, partners)

    arrs = [key, idx]
    for size in [2 << s for s in range(13)]:
        asc = (lin & size) != 0
        d = size // 2
        while d >= 128:
            arrs = row_pass(arrs, asc, d // 128)
            d //= 2
        while d >= 1:
            arrs = lane_pass(arrs, asc, d)
            d //= 2

    base = pl.program_id(0) * (_S * _M) + (r_iota >> 6) * _M
    scores_ref[...] = jax.lax.bitcast_convert_type(arrs[0], jnp.float32)
    gidx_ref[...] = arrs[1] + base


def _gather_body(idx_hbm, t0, t1, t2, t3, o0, o1, o2, o3,
                 idx_v, b0, b1, b2, b3, sem):
    wid = lax.axis_index("s") * 2 + lax.axis_index("c")
    base = wid * _CH
    pltpu.sync_copy(idx_hbm.at[pl.ds(base, _CH)], idx_v)
    cps = [
        pltpu.async_copy(t.at[idx_v], b, sem)
        for t, b in ((t0, b0), (t1, b1), (t2, b2), (t3, b3))
    ]
    for c in cps:
        c.wait()
    for b, o in ((b0, o0), (b1, o1), (b2, o2), (b3, o3)):
        pltpu.sync_copy(b, o.at[pl.ds(base, _CH)])


@functools.partial(jax.jit, static_argnames=())
def kernel(pred_logits, pred_boxes, target_sizes):
    B, N, C = pred_logits.shape
    assert (B, N, C) == (_B, _N, 1)
    nblk = B // _S

    prob = jax.nn.sigmoid(pred_logits.reshape(B, N))
    pad = _M - N
    prob_p = jnp.pad(prob, ((0, 0), (0, pad)), constant_values=-1.0).reshape(nblk, _R, _C)

    cx = jnp.pad(pred_boxes[:, :, 0], ((0, 0), (0, pad))).reshape(nblk, _R, _C)
    cy = jnp.pad(pred_boxes[:, :, 1], ((0, 0), (0, pad))).reshape(nblk, _R, _C)
    w = jnp.pad(pred_boxes[:, :, 2], ((0, 0), (0, pad))).reshape(nblk, _R, _C)
    h = jnp.pad(pred_boxes[:, :, 3], ((0, 0), (0, pad))).reshape(nblk, _R, _C)

    img_h = target_sizes[:, 0].astype(jnp.float32)
    img_w = target_sizes[:, 1].astype(jnp.float32)
    sw = jnp.broadcast_to(img_w[:, None, None], (B, 64, _C)).reshape(nblk, _R, _C)
    sh = jnp.broadcast_to(img_h[:, None, None], (B, 64, _C)).reshape(nblk, _R, _C)

    blk = pl.BlockSpec((1, _R, _C), lambda b: (b, 0, 0))
    out_shape = [
        jax.ShapeDtypeStruct((nblk, _R, _C), jnp.float32),
        jax.ShapeDtypeStruct((nblk, _R, _C), jnp.int32),
        jax.ShapeDtypeStruct((nblk, _R, _C), jnp.float32),
        jax.ShapeDtypeStruct((nblk, _R, _C), jnp.float32),
        jax.ShapeDtypeStruct((nblk, _R, _C), jnp.float32),
        jax.ShapeDtypeStruct((nblk, _R, _C), jnp.float32),
    ]
    scores, gidx, x1, y1, x2, y2 = pl.pallas_call(
        _sort_body,
        grid=(nblk,),
        in_specs=[blk] * 7,
        out_specs=[blk] * 6,
        out_shape=out_shape,
    )(prob_p, cx, cy, w, h, sw, sh)

    flat = (_B * _M,)
    mesh = plsc.VectorSubcoreMesh(core_axis_name="c", subcore_axis_name="s")
    gathered = pl.kernel(
        _gather_body,
        mesh=mesh,
        out_type=[jax.ShapeDtypeStruct(flat, jnp.float32)] * 4,
        scratch_types=[
            pltpu.VMEM((_CH,), jnp.int32),
            pltpu.VMEM((_CH,), jnp.float32),
            pltpu.VMEM((_CH,), jnp.float32),
            pltpu.VMEM((_CH,), jnp.float32),
            pltpu.VMEM((_CH,), jnp.float32),
            pltpu.SemaphoreType.DMA,
        ],
    )(gidx.reshape(flat), x1.reshape(flat), y1.reshape(flat),
      x2.reshape(flat), y2.reshape(flat))

    scores = scores.reshape(B, _M)[:, :_N]
    boxes = jnp.stack(
        [g.reshape(B, _M)[:, :_N] for g in gathered], axis=-1)
    labels = jnp.full((B, N), 1, dtype=jnp.int32)
    return scores, labels, boxes
